# baseline (speedup 1.0000x reference)
import jax
import jax.numpy as jnp
from jax.experimental import pallas as pl

_N = 768
_C = 68
_BK = 4
_BIG = 100000


def _body(ri_ref, cs_ref, rj_ref, cv_ref, out_ref):
    r = pl.program_id(0)
    ri = ri_ref[:, :]
    cs = cs_ref[:, :]
    rj = rj_ref[:, :]
    cv = cv_ref[:, :]

    diff = ri - rj
    same = cs == cv
    e_lo = jnp.where(same, diff, _BIG)
    ones = jnp.ones_like(diff, dtype=jnp.float32)
    zeros = jnp.zeros_like(ones)

    for t in range(_BK):
        k = r * _BK + t
        out_ref[t] = jnp.where(e_lo == k - 32, ones, zeros)

    @pl.when(r == 0)
    def _():
        out_ref[0] = jnp.where(e_lo <= -32, ones, zeros)

    @pl.when(r == 16)
    def _():
        e_hi = jnp.where(same, diff, -_BIG)
        samef = jnp.where(same, ones, zeros)
        out_ref[0] = jnp.where(e_hi >= 32, ones, zeros)
        out_ref[1] = 1.0 - samef
        out_ref[2] = 1.0 - samef
        out_ref[3] = samef


def kernel(residue_index, chain_idx):
    ri = jnp.round(residue_index.reshape(_N)).astype(jnp.int32)
    cv = chain_idx.reshape(_N).astype(jnp.int32)

    ri_col = ri.reshape(_N, 1)
    cs_col = cv.reshape(_N, 1)
    rj_row = ri.reshape(1, _N)
    cv_row = cv.reshape(1, _N)

    grid = (_C // _BK,)
    out = pl.pallas_call(
        _body,
        grid=grid,
        in_specs=[
            pl.BlockSpec((_N, 1), lambda r: (0, 0)),
            pl.BlockSpec((_N, 1), lambda r: (0, 0)),
            pl.BlockSpec((1, _N), lambda r: (0, 0)),
            pl.BlockSpec((1, _N), lambda r: (0, 0)),
        ],
        out_specs=pl.BlockSpec((_BK, _N, _N), lambda r: (r, 0, 0)),
        out_shape=jax.ShapeDtypeStruct((_C, _N, _N), jnp.float32),
    )(ri_col, cs_col, rj_row, cv_row)
    return out.transpose(1, 2, 0).reshape(1, _N, _N, _C)


# speedup vs baseline: 1.0208x; 1.0208x over previous
import jax
import jax.numpy as jnp
from jax.experimental import pallas as pl

_N = 768
_C = 68
_BI = 32
_BIG = 100000


def _body(ri_ref, cs_ref, rj_ref, cv_ref, out_ref):
    ri = ri_ref[:, :]
    cs = cs_ref[:, :]
    rj = rj_ref[:, :]
    cv = cv_ref[:, :]

    diff = ri - rj
    same = cs == cv
    e_lo = jnp.where(same, diff, _BIG)
    e_hi = jnp.where(same, diff, -_BIG)
    ones = jnp.ones_like(diff, dtype=jnp.float32)
    zeros = jnp.zeros_like(ones)
    samef = jnp.where(same, ones, zeros)
    nsf = 1.0 - samef

    out_ref[0] = jnp.where(e_lo <= -32, ones, zeros)
    for k in range(1, 64):
        out_ref[k] = jnp.where(e_lo == k - 32, ones, zeros)
    out_ref[64] = jnp.where(e_hi >= 32, ones, zeros)
    out_ref[65] = nsf
    out_ref[66] = nsf
    out_ref[67] = samef


def kernel(residue_index, chain_idx):
    ri = jnp.round(residue_index.reshape(_N)).astype(jnp.int32)
    cv = chain_idx.reshape(_N).astype(jnp.int32)

    ri_col = ri.reshape(_N, 1)
    cs_col = cv.reshape(_N, 1)
    rj_row = ri.reshape(1, _N)
    cv_row = cv.reshape(1, _N)

    grid = (_N // _BI,)
    out = pl.pallas_call(
        _body,
        grid=grid,
        in_specs=[
            pl.BlockSpec((_BI, 1), lambda r: (r, 0)),
            pl.BlockSpec((_BI, 1), lambda r: (r, 0)),
            pl.BlockSpec((1, _N), lambda r: (0, 0)),
            pl.BlockSpec((1, _N), lambda r: (0, 0)),
        ],
        out_specs=pl.BlockSpec((_C, _BI, _N), lambda r: (0, r, 0)),
        out_shape=jax.ShapeDtypeStruct((_C, _N, _N), jnp.float32),
    )(ri_col, cs_col, rj_row, cv_row)
    return out.transpose(1, 2, 0).reshape(1, _N, _N, _C)
